# baseline (speedup 1.0000x reference)
import functools

import jax
import jax.numpy as jnp
from jax import lax
from jax.experimental import pallas as pl
from jax.experimental.pallas import tpu as pltpu
from jax.experimental.pallas import tpu_sc as plsc

_NC = 2
_NS = 16
_NW = _NC * _NS
_L = 16


@functools.partial(jax.jit, static_argnames=("total_chunks", "c_nodes", "s"))
def _sc_mean_gather(idx, table, *, total_chunks, c_nodes, s):
    rows = c_nodes * s
    _, d = table.shape
    b_out = total_chunks * c_nodes
    nbuf = 4
    t_max = -(-total_chunks // _NW)
    t_max += (-t_max) % nbuf
    scale = jnp.float32(1.0 / (float(s) + 1e-15))

    mesh = plsc.VectorSubcoreMesh(core_axis_name="c", subcore_axis_name="s",
                                  num_cores=_NC, num_subcores=_NS)

    @functools.partial(
        pl.kernel,
        out_type=jax.ShapeDtypeStruct((b_out, d), jnp.float32),
        mesh=mesh,
        scratch_types=(
            [pltpu.VMEM((t_max * rows,), jnp.int32)]
            + [pltpu.VMEM((rows, d), jnp.float32) for _ in range(nbuf)]
            + [pltpu.VMEM((c_nodes, d), jnp.float32) for _ in range(nbuf)]
            + [pltpu.SemaphoreType.DMA for _ in range(2 * nbuf)]
        ),
    )
    def k(idx_hbm, table_hbm, out_hbm, idx_v, *bufs):
        rows_v = bufs[:nbuf]
        out_v = bufs[nbuf:2 * nbuf]
        gsem = bufs[2 * nbuf:3 * nbuf]
        osem = bufs[3 * nbuf:4 * nbuf]

        wid = lax.axis_index("c") * _NS + lax.axis_index("s")
        start_w = (wid * total_chunks) // _NW
        n_w = ((wid + 1) * total_chunks) // _NW - start_w
        nm1 = n_w - 1

        pltpu.sync_copy(idx_hbm.at[pl.ds(start_w * rows, t_max * rows)],
                        idx_v)

        def gather(step_lc, b):
            return pltpu.make_async_copy(
                table_hbm.at[idx_v.at[pl.ds(step_lc * rows, rows)]],
                rows_v[b], gsem[b])

        def out_store(step_lc, b):
            return pltpu.make_async_copy(
                out_v[b],
                out_hbm.at[pl.ds((start_w + step_lc) * c_nodes, c_nodes)],
                osem[b])

        for b in range(nbuf):
            gather(lax.min(jnp.int32(b), nm1), b).start()

        @pl.loop(0, t_max // nbuf)
        def _steps(t):
            for b in range(nbuf):
                i = nbuf * t + b
                lc = lax.min(i, nm1)
                gather(lc, b).wait()

                @pl.when(t >= 1)
                def _():
                    out_store(lax.min(i - nbuf, nm1), b).wait()

                rv, ov = rows_v[b], out_v[b]
                for n in range(c_nodes):
                    for c in range(d // _L):
                        sl = pl.ds(c * _L, _L)
                        acc = rv[n * s, sl]
                        for kk in range(1, s):
                            acc = acc + rv[n * s + kk, sl]
                        ov[n, sl] = acc * scale

                out_store(lc, b).start()
                gather(lax.min(i + nbuf, nm1), b).start()

        for b in range(nbuf):
            gather(nm1, b).wait()
            out_store(nm1, b).wait()

    return k(idx, table)


def kernel(nodes, to_neighs, table):
    b, s = to_neighs.shape
    c_nodes = 8
    total_chunks = -(-b // c_nodes)
    idx = to_neighs.reshape(-1)
    if total_chunks * c_nodes != b:
        idx = jnp.pad(idx, (0, (total_chunks * c_nodes - b) * s))
    t_max = -(-total_chunks // _NW)
    t_max += (-t_max) % 4
    needed = (((_NW - 1) * total_chunks) // _NW + t_max) * c_nodes * s
    if needed > idx.shape[0]:
        idx = jnp.pad(idx, (0, needed - idx.shape[0]))
    out = _sc_mean_gather(idx, table, total_chunks=total_chunks,
                          c_nodes=c_nodes, s=s)
    return out[:b] if total_chunks * c_nodes != b else out
